# baseline (speedup 1.0000x reference)
import functools

import jax
import jax.numpy as jnp
from jax.experimental import pallas as pl
from jax.experimental.pallas import tpu as pltpu

N = 100000
D = 128
SCALE = 100.0


def _flash_body(x_ref, k_ref, v_ref, o_ref, acc_ref, m_ref, l_ref, *, nblocks):
    j = pl.program_id(1)

    @pl.when(j == 0)
    def _init():
        m_ref[...] = jnp.full_like(m_ref, -jnp.inf)
        l_ref[...] = jnp.zeros_like(l_ref)
        acc_ref[...] = jnp.zeros_like(acc_ref)

    s = jax.lax.dot_general(
        x_ref[...], k_ref[...], (((1,), (1,)), ((), ())),
        preferred_element_type=jnp.float32,
    ) * SCALE

    m_prev = m_ref[...][:, :1]
    l_prev = l_ref[...][:, :1]
    m_cur = jnp.max(s, axis=1, keepdims=True)
    m_new = jnp.maximum(m_prev, m_cur)
    alpha = jnp.exp(m_prev - m_new)
    p = jnp.exp(s - m_new)
    l_new = alpha * l_prev + jnp.sum(p, axis=1, keepdims=True)
    pv = jax.lax.dot_general(
        p.astype(jnp.bfloat16), v_ref[...], (((1,), (0,)), ((), ())),
        preferred_element_type=jnp.float32,
    )
    acc_ref[...] = acc_ref[...] * alpha + pv
    m_ref[...] = jnp.broadcast_to(m_new, m_ref.shape)
    l_ref[...] = jnp.broadcast_to(l_new, l_ref.shape)

    @pl.when(j == nblocks - 1)
    def _finish():
        o_ref[...] = acc_ref[...] / l_ref[...][:, :1]


@jax.jit
def kernel(x, keys, vals):
    bq = 256
    bn = 800
    nq = x.shape[0] // bq
    nblocks = N // bn

    xb = x.astype(jnp.bfloat16)
    kb = keys.astype(jnp.bfloat16)
    vb = vals.astype(jnp.bfloat16)

    return pl.pallas_call(
        functools.partial(_flash_body, nblocks=nblocks),
        grid=(nq, nblocks),
        in_specs=[
            pl.BlockSpec((bq, D), lambda i, j: (i, 0)),
            pl.BlockSpec((bn, D), lambda i, j: (j, 0)),
            pl.BlockSpec((bn, D), lambda i, j: (j, 0)),
        ],
        out_specs=pl.BlockSpec((bq, D), lambda i, j: (i, 0)),
        out_shape=jax.ShapeDtypeStruct((x.shape[0], D), jnp.float32),
        scratch_shapes=[
            pltpu.VMEM((bq, D), jnp.float32),
            pltpu.VMEM((bq, D), jnp.float32),
            pltpu.VMEM((bq, D), jnp.float32),
        ],
    )(xb, kb, vb)


# speedup vs baseline: 1.3689x; 1.3689x over previous
import functools
import math

import jax
import jax.numpy as jnp
from jax.experimental import pallas as pl
from jax.experimental.pallas import tpu as pltpu

N = 100000
D = 128
SCALE = 100.0
C = SCALE * math.log2(math.e)
DV = D + 8


def _flash_body(x_ref, k_ref, v_ref, o_ref, acc_ref, m_ref, *, nblocks):
    j = pl.program_id(1)

    @pl.when(j == 0)
    def _init():
        m_ref[...] = jnp.full_like(m_ref, -jnp.inf)
        acc_ref[...] = jnp.zeros_like(acc_ref)

    s = jax.lax.dot_general(
        x_ref[...], k_ref[...], (((1,), (1,)), ((), ())),
        preferred_element_type=jnp.float32,
    )

    m_prev = m_ref[...][:, :1]
    m_cur = jnp.max(s, axis=1, keepdims=True)
    m_new = jnp.maximum(m_prev, m_cur)
    alpha = jnp.exp2((m_prev - m_new) * C)
    p = jnp.exp2((s - m_new) * C).astype(jnp.bfloat16)
    pv = jax.lax.dot_general(
        p, v_ref[...], (((1,), (0,)), ((), ())),
        preferred_element_type=jnp.float32,
    )
    acc_ref[...] = acc_ref[...] * alpha + pv
    m_ref[...] = jnp.broadcast_to(m_new, m_ref.shape)

    @pl.when(j == nblocks - 1)
    def _finish():
        acc = acc_ref[...]
        o_ref[...] = acc[:, :D] / acc[:, D:D + 1]


@jax.jit
def kernel(x, keys, vals):
    bq = 256
    bn = 2000
    nq = x.shape[0] // bq
    nblocks = N // bn

    xb = x.astype(jnp.bfloat16)
    kb = keys.astype(jnp.bfloat16)
    vb = jnp.concatenate(
        [vals.astype(jnp.bfloat16),
         jnp.ones((N, DV - D), dtype=jnp.bfloat16)], axis=1)

    return pl.pallas_call(
        functools.partial(_flash_body, nblocks=nblocks),
        grid=(nq, nblocks),
        in_specs=[
            pl.BlockSpec((bq, D), lambda i, j: (i, 0)),
            pl.BlockSpec((bn, D), lambda i, j: (j, 0)),
            pl.BlockSpec((bn, DV), lambda i, j: (j, 0)),
        ],
        out_specs=pl.BlockSpec((bq, D), lambda i, j: (i, 0)),
        out_shape=jax.ShapeDtypeStruct((x.shape[0], D), jnp.float32),
        scratch_shapes=[
            pltpu.VMEM((bq, DV), jnp.float32),
            pltpu.VMEM((bq, D), jnp.float32),
        ],
    )(xb, kb, vb)


# speedup vs baseline: 1.7165x; 1.2540x over previous
import functools
import math

import jax
import jax.numpy as jnp
from jax.experimental import pallas as pl
from jax.experimental.pallas import tpu as pltpu

N = 100000
D = 128
SCALE = 100.0
C = SCALE * math.log2(math.e)
FILL = -3e38


def _body(x_ref, k_ref, v_ref, o_ref, s_scr, acc_ref, m_ref, l_ref, *, nb):
    j = pl.program_id(1)
    buf = jax.lax.rem(j, 2)

    @pl.when(j == 0)
    def _init():
        m_ref[...] = jnp.full_like(m_ref, -jnp.inf)
        l_ref[...] = jnp.zeros_like(l_ref)
        acc_ref[...] = jnp.zeros_like(acc_ref)
        s_scr[1, :, :] = jnp.full(s_scr.shape[1:], FILL, s_scr.dtype)

    sp = s_scr[1 - buf, :, :]
    m_prev = m_ref[...][:, :1]
    m_cur = jnp.max(sp, axis=1, keepdims=True)
    m_new = jnp.maximum(m_prev, m_cur)
    alpha = jnp.exp2((m_prev - m_new) * C)
    p = jnp.exp2((sp - m_new) * C)
    l_ref[...] = l_ref[...] * alpha + jnp.broadcast_to(
        jnp.sum(p, axis=1, keepdims=True), l_ref.shape)
    pv = jax.lax.dot_general(
        p.astype(jnp.bfloat16), v_ref[...], (((1,), (0,)), ((), ())),
        preferred_element_type=jnp.float32,
    )
    acc_ref[...] = acc_ref[...] * alpha + pv
    m_ref[...] = jnp.broadcast_to(m_new, m_ref.shape)

    s = jax.lax.dot_general(
        x_ref[...], k_ref[...], (((1,), (1,)), ((), ())),
        preferred_element_type=jnp.float32,
    )
    s_scr[buf, :, :] = s

    @pl.when(j == nb)
    def _finish():
        o_ref[...] = acc_ref[...] / l_ref[...][:, :1]


@jax.jit
def kernel(x, keys, vals):
    bq = 256
    bn = 2000
    nq = x.shape[0] // bq
    nb = N // bn

    xb = x.astype(jnp.bfloat16)
    kb = keys.astype(jnp.bfloat16)
    vb = vals.astype(jnp.bfloat16)

    return pl.pallas_call(
        functools.partial(_body, nb=nb),
        grid=(nq, nb + 1),
        in_specs=[
            pl.BlockSpec((bq, D), lambda i, j: (i, 0)),
            pl.BlockSpec((bn, D), lambda i, j, nb=nb: (jax.lax.rem(j, nb), 0)),
            pl.BlockSpec((bn, D),
                         lambda i, j, nb=nb: (jax.lax.rem(j + nb - 1, nb), 0)),
        ],
        out_specs=pl.BlockSpec((bq, D), lambda i, j: (i, 0)),
        out_shape=jax.ShapeDtypeStruct((x.shape[0], D), jnp.float32),
        scratch_shapes=[
            pltpu.VMEM((2, bq, bn), jnp.float32),
            pltpu.VMEM((bq, D), jnp.float32),
            pltpu.VMEM((bq, D), jnp.float32),
            pltpu.VMEM((bq, D), jnp.float32),
        ],
    )(xb, kb, vb)


# speedup vs baseline: 2.1165x; 1.2330x over previous
import functools
import math

import jax
import jax.numpy as jnp
from jax.experimental import pallas as pl
from jax.experimental.pallas import tpu as pltpu

N = 100000
D = 128
SCALE = 100.0
C = SCALE * math.log2(math.e)


def _body(x_ref, k_ref, v_ref, acc_ref, l_ref, s_scr, m_ref, *, nb):
    j = pl.program_id(1)
    buf = jax.lax.rem(j, 2)

    @pl.when(j == 0)
    def _init():
        m_ref[...] = jnp.full_like(m_ref, -jnp.inf)
        l_ref[...] = jnp.zeros_like(l_ref)
        acc_ref[...] = jnp.zeros_like(acc_ref)

    @pl.when(j > 0)
    def _process():
        sp = s_scr[1 - buf, :, :]
        m_prev = m_ref[...][:, :1]
        m_cur = jnp.max(sp, axis=1, keepdims=True)
        m_new = jnp.maximum(m_prev, m_cur)
        alpha = jnp.exp2((m_prev - m_new) * C)
        p = jnp.exp2((sp - m_new) * C)
        l_ref[...] = l_ref[...] * alpha + jnp.broadcast_to(
            jnp.sum(p, axis=1, keepdims=True), l_ref.shape)
        pv = jax.lax.dot_general(
            p.astype(jnp.bfloat16), v_ref[...], (((1,), (0,)), ((), ())),
            preferred_element_type=jnp.float32,
        )
        acc_ref[...] = acc_ref[...] * alpha + pv
        m_ref[...] = jnp.broadcast_to(m_new, m_ref.shape)

    @pl.when(j < nb)
    def _produce():
        s = jax.lax.dot_general(
            x_ref[...], k_ref[...], (((1,), (1,)), ((), ())),
            preferred_element_type=jnp.float32,
        )
        s_scr[buf, :, :] = s


@jax.jit
def kernel(x, keys, vals):
    bq = 1024
    bn = 2000
    nq = x.shape[0] // bq
    nb = N // bn

    xb = x.astype(jnp.bfloat16)
    kb = keys.astype(jnp.bfloat16)
    vb = vals.astype(jnp.bfloat16)

    acc, l = pl.pallas_call(
        functools.partial(_body, nb=nb),
        grid=(nq, nb + 1),
        in_specs=[
            pl.BlockSpec((bq, D), lambda i, j: (i, 0)),
            pl.BlockSpec((bn, D), lambda i, j, nb=nb: (jax.lax.rem(j, nb), 0)),
            pl.BlockSpec((bn, D),
                         lambda i, j, nb=nb: (jax.lax.rem(j + nb - 1, nb), 0)),
        ],
        out_specs=[
            pl.BlockSpec((bq, D), lambda i, j: (i, 0)),
            pl.BlockSpec((bq, D), lambda i, j: (i, 0)),
        ],
        out_shape=[
            jax.ShapeDtypeStruct((x.shape[0], D), jnp.float32),
            jax.ShapeDtypeStruct((x.shape[0], D), jnp.float32),
        ],
        compiler_params=pltpu.CompilerParams(
            dimension_semantics=("parallel", "arbitrary")),
        scratch_shapes=[
            pltpu.VMEM((2, bq, bn), jnp.float32),
            pltpu.VMEM((bq, D), jnp.float32),
        ],
    )(xb, kb, vb)
    return acc / l
